# baseline (speedup 1.0000x reference)
import functools

import jax
import jax.numpy as jnp
from jax import lax
from jax.experimental import pallas as pl
from jax.experimental.pallas import tpu as pltpu
from jax.experimental.pallas import tpu_sc as plsc

NUM_FIELDS = 26
VOCAB = 100000
EMB_DIM = 16
BATCH = 16384

NC = 2
NS = 16
NW = NC * NS
ROWS_PER_W = BATCH // NW
P_ROWS = 128
N_PASS = ROWS_PER_W // P_ROWS


def _build_sc_gather():
    mesh = plsc.VectorSubcoreMesh(core_axis_name="c", subcore_axis_name="s")

    @functools.partial(
        pl.kernel,
        mesh=mesh,
        compiler_params=pltpu.CompilerParams(use_tc_tiling_on_sc=False,
                                             needs_layout_passes=False),
        out_type=jax.ShapeDtypeStruct((NUM_FIELDS * EMB_DIM, BATCH),
                                      jnp.float32),
        scratch_types=[
            pltpu.VMEM((NUM_FIELDS, EMB_DIM, P_ROWS), jnp.int32),
            pltpu.VMEM((2, EMB_DIM, P_ROWS), jnp.float32),
            pltpu.SemaphoreType.DMA,
            pltpu.SemaphoreType.DMA,
        ],
    )
    def k(xt_hbm, tabt_hbm, out_hbm, idx_v, blk_v, gsem, wsem):
        wid = lax.axis_index("s") * NC + lax.axis_index("c")

        def one_pass(p, carry):
            b0 = wid * ROWS_PER_W + p * P_ROWS
            pltpu.sync_copy(xt_hbm.at[:, :, pl.ds(b0, P_ROWS)], idx_v)

            def one_field(f, c2):
                slot = lax.rem(f, 2)
                u = p * NUM_FIELDS + f

                @pl.when(u >= 2)
                def _():
                    pltpu.make_async_copy(
                        blk_v.at[slot],
                        out_hbm.at[pl.ds(0, EMB_DIM), pl.ds(0, P_ROWS)],
                        wsem).wait()

                gathers = []
                for c in range(EMB_DIM):
                    gathers.append(pltpu.async_copy(
                        tabt_hbm.at[f].at[idx_v.at[f, c]],
                        blk_v.at[slot, c],
                        gsem))
                for g in gathers:
                    g.wait()

                pltpu.async_copy(
                    blk_v.at[slot],
                    out_hbm.at[pl.ds(f * EMB_DIM, EMB_DIM),
                               pl.ds(b0, P_ROWS)],
                    wsem)
                return c2

            lax.fori_loop(0, NUM_FIELDS, one_field, 0)
            return carry

        lax.fori_loop(0, N_PASS, one_pass, 0)

        for slot in range(2):
            pltpu.make_async_copy(
                blk_v.at[slot],
                out_hbm.at[pl.ds(0, EMB_DIM), pl.ds(0, P_ROWS)],
                wsem).wait()

    return k


def kernel(x, tables):
    xoff = (x * EMB_DIM).T[:, None, :] + jnp.arange(
        EMB_DIM, dtype=jnp.int32)[None, :, None]
    tab_flat = tables.reshape(NUM_FIELDS, VOCAB * EMB_DIM)
    out_t = _build_sc_gather()(xoff, tab_flat)
    return out_t.T


# speedup vs baseline: 2.2640x; 2.2640x over previous
import functools

import jax
import jax.numpy as jnp
from jax import lax
from jax.experimental import pallas as pl
from jax.experimental.pallas import tpu as pltpu
from jax.experimental.pallas import tpu_sc as plsc

NUM_FIELDS = 26
VOCAB = 100000
EMB_DIM = 16
BATCH = 16384

NC = 2
NS = 16
NW = NC * NS
ROWS_PER_W = BATCH // NW
P_ROWS = 128
N_PASS = ROWS_PER_W // P_ROWS

NF_A = 13
NF_B = NUM_FIELDS - NF_A


def _build_sc_gather(nf):
    mesh = plsc.VectorSubcoreMesh(core_axis_name="c", subcore_axis_name="s")

    @functools.partial(
        pl.kernel,
        mesh=mesh,
        compiler_params=pltpu.CompilerParams(use_tc_tiling_on_sc=False,
                                             needs_layout_passes=False),
        out_type=jax.ShapeDtypeStruct((nf * EMB_DIM, BATCH), jnp.float32),
        scratch_types=[
            pltpu.VMEM((nf, P_ROWS), jnp.int32),
            pltpu.VMEM((2, EMB_DIM, P_ROWS), jnp.float32),
            pltpu.SemaphoreType.DMA,
            pltpu.SemaphoreType.DMA,
        ],
    )
    def k(xt_hbm, tabt_hbm, out_hbm, idx_v, blk_v, gsem, wsem):
        wid = lax.axis_index("s") * NC + lax.axis_index("c")

        def one_pass(p, carry):
            b0 = wid * ROWS_PER_W + p * P_ROWS
            pltpu.sync_copy(xt_hbm.at[:, pl.ds(b0, P_ROWS)], idx_v)

            def one_field(f, c2):
                slot = lax.rem(f, 2)
                u = p * nf + f

                @pl.when(u >= 2)
                def _():
                    pltpu.make_async_copy(
                        blk_v.at[slot],
                        out_hbm.at[pl.ds(0, EMB_DIM), pl.ds(0, P_ROWS)],
                        wsem).wait()

                gathers = []
                for c in range(EMB_DIM):
                    gathers.append(pltpu.async_copy(
                        tabt_hbm.at[f, c].at[idx_v.at[f]],
                        blk_v.at[slot, c],
                        gsem))
                for g in gathers:
                    g.wait()

                pltpu.async_copy(
                    blk_v.at[slot],
                    out_hbm.at[pl.ds(f * EMB_DIM, EMB_DIM),
                               pl.ds(b0, P_ROWS)],
                    wsem)
                return c2

            lax.fori_loop(0, nf, one_field, 0)
            return carry

        lax.fori_loop(0, N_PASS, one_pass, 0)

        for slot in range(2):
            pltpu.make_async_copy(
                blk_v.at[slot],
                out_hbm.at[pl.ds(0, EMB_DIM), pl.ds(0, P_ROWS)],
                wsem).wait()

    return k


def kernel(x, tables):
    xt = x.T
    tabt_a = jnp.swapaxes(tables[:NF_A], 1, 2)
    tabt_b = jnp.swapaxes(tables[NF_A:], 1, 2)
    out_a = _build_sc_gather(NF_A)(xt[:NF_A], tabt_a)
    out_b = _build_sc_gather(NF_B)(xt[NF_A:], tabt_b)
    return jnp.concatenate([out_a.T, out_b.T], axis=1)
